# baseline (speedup 1.0000x reference)
import functools
import jax
import jax.numpy as jnp
from jax.experimental import pallas as pl
from jax.experimental.pallas import tpu as pltpu

B, S, D = 2, 2048, 1024
E, K, DFF = 8, 2, 512
SHF = 1024
T = B * S
TB = 512


def _moe_body(x_ref, gw_ref, eg_ref, eu_ref, ed_ref, sg_ref, su_ref, sd_ref,
              o_ref):
    x32 = x_ref[...]
    x = x32.astype(jnp.bfloat16)

    logits = jax.lax.dot_general(
        x, gw_ref[...].astype(jnp.bfloat16), (((1,), (0,)), ((), ())),
        preferred_element_type=jnp.float32)
    m = jnp.max(logits, axis=-1, keepdims=True)
    ex = jnp.exp(logits - m)
    scores = ex / jnp.sum(ex, axis=-1, keepdims=True)
    lane = jax.lax.broadcasted_iota(jnp.int32, scores.shape, 1)
    m1 = jnp.max(scores, axis=-1, keepdims=True)
    eq1 = (scores == m1)
    i1 = jnp.min(jnp.where(eq1, lane, E), axis=-1, keepdims=True)
    msk1 = (lane == i1)
    s2 = jnp.where(msk1, -jnp.inf, scores)
    m2 = jnp.max(s2, axis=-1, keepdims=True)
    eq2 = (s2 == m2)
    i2 = jnp.min(jnp.where(eq2, lane, E), axis=-1, keepdims=True)
    msk2 = (lane == i2)
    denom = m1 + m2 + 1e-20
    comb = (jnp.where(msk1, scores, 0.0) + jnp.where(msk2, scores, 0.0)) / denom

    acc = jnp.zeros((TB, D), jnp.float32)
    for e in range(E):
        g = jax.lax.dot_general(x, eg_ref[e], (((1,), (0,)), ((), ())),
                                preferred_element_type=jnp.float32)
        u = jax.lax.dot_general(x, eu_ref[e], (((1,), (0,)), ((), ())),
                                preferred_element_type=jnp.float32)
        h = (g * jax.nn.sigmoid(g)) * u
        hw = (h * comb[:, e:e + 1]).astype(jnp.bfloat16)
        acc = acc + jax.lax.dot_general(hw, ed_ref[e], (((1,), (0,)), ((), ())),
                                        preferred_element_type=jnp.float32)
    g = jax.lax.dot_general(x, sg_ref[...], (((1,), (0,)), ((), ())),
                            preferred_element_type=jnp.float32)
    u = jax.lax.dot_general(x, su_ref[...], (((1,), (0,)), ((), ())),
                            preferred_element_type=jnp.float32)
    h = ((g * jax.nn.sigmoid(g)) * u).astype(jnp.bfloat16)
    acc = acc + jax.lax.dot_general(h, sd_ref[...], (((1,), (0,)), ((), ())),
                                    preferred_element_type=jnp.float32)
    o_ref[...] = acc


@jax.jit
def kernel(hidden_states, gate_w, expert_gate, expert_up, expert_down,
           shared_gate, shared_up, shared_down):
    x = hidden_states.reshape(T, D)
    gw_t = gate_w.T
    eg_t = expert_gate.transpose(0, 2, 1).astype(jnp.bfloat16)
    eu_t = expert_up.transpose(0, 2, 1).astype(jnp.bfloat16)
    ed_t = expert_down.transpose(0, 2, 1).astype(jnp.bfloat16)
    sg_t = shared_gate.T.astype(jnp.bfloat16)
    su_t = shared_up.T.astype(jnp.bfloat16)
    sd_t = shared_down.T.astype(jnp.bfloat16)

    grid = (T // TB,)
    out = pl.pallas_call(
        _moe_body,
        grid=grid,
        in_specs=[
            pl.BlockSpec((TB, D), lambda i: (i, 0)),
            pl.BlockSpec((D, E), lambda i: (0, 0)),
            pl.BlockSpec((E, D, DFF), lambda i: (0, 0, 0)),
            pl.BlockSpec((E, D, DFF), lambda i: (0, 0, 0)),
            pl.BlockSpec((E, DFF, D), lambda i: (0, 0, 0)),
            pl.BlockSpec((D, SHF), lambda i: (0, 0)),
            pl.BlockSpec((D, SHF), lambda i: (0, 0)),
            pl.BlockSpec((SHF, D), lambda i: (0, 0)),
        ],
        out_specs=pl.BlockSpec((TB, D), lambda i: (i, 0)),
        out_shape=jax.ShapeDtypeStruct((T, D), jnp.float32),
        compiler_params=pltpu.CompilerParams(
            dimension_semantics=("arbitrary",),
        ),
    )(x, gw_t, eg_t, eu_t, ed_t, sg_t, su_t, sd_t)
    return out.reshape(B, S, D)


# speedup vs baseline: 1.0201x; 1.0201x over previous
import functools
import jax
import jax.numpy as jnp
from jax.experimental import pallas as pl
from jax.experimental.pallas import tpu as pltpu

B, S, D = 2, 2048, 1024
E, K, DFF = 8, 2, 512
SHF = 1024
T = B * S
TB = 1024


def _moe_body(x_ref, gw_ref, eg_ref, eu_ref, ed_ref, sg_ref, su_ref, sd_ref,
              o_ref):
    x32 = x_ref[...]
    x = x32.astype(jnp.bfloat16)

    logits = jax.lax.dot_general(
        x, gw_ref[...].astype(jnp.bfloat16), (((1,), (0,)), ((), ())),
        preferred_element_type=jnp.float32)
    m = jnp.max(logits, axis=-1, keepdims=True)
    ex = jnp.exp(logits - m)
    scores = ex / jnp.sum(ex, axis=-1, keepdims=True)
    lane = jax.lax.broadcasted_iota(jnp.int32, scores.shape, 1)
    m1 = jnp.max(scores, axis=-1, keepdims=True)
    eq1 = (scores == m1)
    i1 = jnp.min(jnp.where(eq1, lane, E), axis=-1, keepdims=True)
    msk1 = (lane == i1)
    s2 = jnp.where(msk1, -jnp.inf, scores)
    m2 = jnp.max(s2, axis=-1, keepdims=True)
    eq2 = (s2 == m2)
    i2 = jnp.min(jnp.where(eq2, lane, E), axis=-1, keepdims=True)
    msk2 = (lane == i2)
    denom = m1 + m2 + 1e-20
    comb = (jnp.where(msk1, scores, 0.0) + jnp.where(msk2, scores, 0.0)) / denom

    acc = jnp.zeros((TB, D), jnp.float32)
    for e in range(E):
        g = jax.lax.dot_general(x, eg_ref[e], (((1,), (0,)), ((), ())),
                                preferred_element_type=jnp.float32)
        u = jax.lax.dot_general(x, eu_ref[e], (((1,), (0,)), ((), ())),
                                preferred_element_type=jnp.float32)
        h = (g * jax.nn.sigmoid(g)) * u
        hw = (h * comb[:, e:e + 1]).astype(jnp.bfloat16)
        acc = acc + jax.lax.dot_general(hw, ed_ref[e], (((1,), (0,)), ((), ())),
                                        preferred_element_type=jnp.float32)
    g = jax.lax.dot_general(x, sg_ref[...], (((1,), (0,)), ((), ())),
                            preferred_element_type=jnp.float32)
    u = jax.lax.dot_general(x, su_ref[...], (((1,), (0,)), ((), ())),
                            preferred_element_type=jnp.float32)
    h = ((g * jax.nn.sigmoid(g)) * u).astype(jnp.bfloat16)
    acc = acc + jax.lax.dot_general(h, sd_ref[...], (((1,), (0,)), ((), ())),
                                    preferred_element_type=jnp.float32)
    o_ref[...] = acc


@jax.jit
def kernel(hidden_states, gate_w, expert_gate, expert_up, expert_down,
           shared_gate, shared_up, shared_down):
    x = hidden_states.reshape(T, D)
    gw_t = gate_w.T
    eg_t = expert_gate.transpose(0, 2, 1).astype(jnp.bfloat16)
    eu_t = expert_up.transpose(0, 2, 1).astype(jnp.bfloat16)
    ed_t = expert_down.transpose(0, 2, 1).astype(jnp.bfloat16)
    sg_t = shared_gate.T.astype(jnp.bfloat16)
    su_t = shared_up.T.astype(jnp.bfloat16)
    sd_t = shared_down.T.astype(jnp.bfloat16)

    grid = (T // TB,)
    out = pl.pallas_call(
        _moe_body,
        grid=grid,
        in_specs=[
            pl.BlockSpec((TB, D), lambda i: (i, 0)),
            pl.BlockSpec((D, E), lambda i: (0, 0)),
            pl.BlockSpec((E, D, DFF), lambda i: (0, 0, 0)),
            pl.BlockSpec((E, D, DFF), lambda i: (0, 0, 0)),
            pl.BlockSpec((E, DFF, D), lambda i: (0, 0, 0)),
            pl.BlockSpec((D, SHF), lambda i: (0, 0)),
            pl.BlockSpec((D, SHF), lambda i: (0, 0)),
            pl.BlockSpec((SHF, D), lambda i: (0, 0)),
        ],
        out_specs=pl.BlockSpec((TB, D), lambda i: (i, 0)),
        out_shape=jax.ShapeDtypeStruct((T, D), jnp.float32),
        compiler_params=pltpu.CompilerParams(
            dimension_semantics=("arbitrary",),
        ),
    )(x, gw_t, eg_t, eu_t, ed_t, sg_t, su_t, sd_t)
    return out.reshape(B, S, D)
